# Initial kernel scaffold; baseline (speedup 1.0000x reference)
import jax
import jax.numpy as jnp
from jax.experimental import pallas as pl


def kernel(pointclouds, valid_points):
    raise NotImplementedError("write your pallas kernel here")


# speedup vs baseline: 1.4050x; 1.4050x over previous
import functools

import jax
import jax.numpy as jnp
from jax import lax
from jax.experimental import pallas as pl
from jax.experimental.pallas import tpu as pltpu
from jax.experimental.pallas import tpu_sc as plsc

_B = 16
_NPTS = 131072
_RES = 50
_NVOX = _RES * _RES * _RES
_NVOX_PAD = 125008
_OUT_W = _NVOX * 4

_CHUNK_PTS = 256
_PT_W = _CHUNK_PTS * 6
_N_CHUNKS = _NPTS // _CHUNK_PTS
_GRP_PER_CHUNK = _CHUNK_PTS // 16

_EXP_VOX = 512
_EXP_W = _EXP_VOX * 4
_N_EXP_FULL = _NVOX // _EXP_VOX
_TAIL_VOX = _NVOX - _N_EXP_FULL * _EXP_VOX
_TAIL_GRPS = (_NVOX_PAD - _N_EXP_FULL * _EXP_VOX) // 16
_TAIL_W = _TAIL_VOX * 4


def _voxelize_body(pts_hbm, val_hbm, out_hbm, pts_v, val_v, occ_v, exp_v):
    c = lax.axis_index("c")
    s = lax.axis_index("s")

    iota = lax.iota(jnp.int32, 16)
    iota6 = iota * 6
    iota4 = iota * 4
    ones = jnp.full((16,), 1.0, jnp.float32)
    zeros = jnp.zeros((16,), jnp.float32)

    @pl.when(s < 8)
    def _():
        b = c * 8 + s

        @pl.loop(0, _NVOX_PAD // 16, unroll=8)
        def _zero(i):
            occ_v[pl.ds(i * 16, 16)] = zeros

        @pl.loop(0, _N_CHUNKS)
        def _chunk(ci):
            pltpu.sync_copy(
                pts_hbm.at[pl.ds(b * (_NPTS * 6) + ci * _PT_W, _PT_W)], pts_v)
            pltpu.sync_copy(
                val_hbm.at[pl.ds(b * _NPTS + ci * _CHUNK_PTS, _CHUNK_PTS)],
                val_v)

            @pl.loop(0, _GRP_PER_CHUNK)
            def _grp(g):
                fbase = g * 96
                gx = iota6 + fbase
                x = plsc.load_gather(pts_v, [gx])
                y = plsc.load_gather(pts_v, [gx + 1])
                z = plsc.load_gather(pts_v, [gx + 2])
                vf = val_v[pl.ds(g * 16, 16)]
                msk = vf > 0.0

                def vox(p):
                    t = ((p + 2.0) * 0.25) * 49.0
                    t = jnp.minimum(jnp.maximum(t, 0.0), 49.0)
                    return t.astype(jnp.int32)

                v = (vox(x) * 50 + vox(y)) * 50 + vox(z)
                plsc.store_scatter(occ_v, [v], ones, mask=msk)

        @pl.loop(0, _EXP_W // 16, unroll=8)
        def _zexp(i):
            exp_v[pl.ds(i * 16, 16)] = zeros

        out_base = b * _OUT_W

        @pl.loop(0, _N_EXP_FULL)
        def _exp(e):
            vbase = e * _EXP_VOX

            @pl.loop(0, _EXP_VOX // 16)
            def _egrp(g):
                occv = occ_v[pl.ds(vbase + g * 16, 16)]
                plsc.store_scatter(exp_v, [iota4 + g * 64], occv)

            pltpu.sync_copy(exp_v,
                            out_hbm.at[pl.ds(out_base + e * _EXP_W, _EXP_W)])

        tbase = _N_EXP_FULL * _EXP_VOX

        @pl.loop(0, _TAIL_GRPS)
        def _tgrp(g):
            occv = occ_v[pl.ds(tbase + g * 16, 16)]
            plsc.store_scatter(exp_v, [iota4 + g * 64], occv)

        pltpu.sync_copy(
            exp_v.at[pl.ds(0, _TAIL_W)],
            out_hbm.at[pl.ds(out_base + tbase * 4, _TAIL_W)])


@functools.partial(jax.jit, static_argnames=())
def _voxelize(pts_flat, val_flat):
    mesh = plsc.VectorSubcoreMesh(core_axis_name="c", subcore_axis_name="s")
    return pl.kernel(
        _voxelize_body,
        out_type=jax.ShapeDtypeStruct((_B * _OUT_W,), jnp.float32),
        mesh=mesh,
        compiler_params=pltpu.CompilerParams(needs_layout_passes=False),
        scratch_types=[
            pltpu.VMEM((_PT_W,), jnp.float32),
            pltpu.VMEM((_CHUNK_PTS,), jnp.float32),
            pltpu.VMEM((_NVOX_PAD,), jnp.float32),
            pltpu.VMEM((_EXP_W,), jnp.float32),
        ],
    )(pts_flat, val_flat)


def kernel(pointclouds, valid_points):
    pts_flat = pointclouds.reshape(-1)
    val_flat = valid_points.astype(jnp.float32).reshape(-1)
    out = _voxelize(pts_flat, val_flat)
    return out.reshape(_B, _RES, _RES, _RES, 4)


# speedup vs baseline: 1.7479x; 1.2440x over previous
import jax
import jax.numpy as jnp
from jax import lax
from jax.experimental import pallas as pl
from jax.experimental.pallas import tpu as pltpu
from jax.experimental.pallas import tpu_sc as plsc

_B = 16
_NPTS = 131072
_RES = 50
_NVOX = _RES * _RES * _RES
_NVOX_PAD = 125024
_HALF_VOX = _NVOX_PAD // 2
_OUT_W = _NVOX * 4

_CHUNK_PTS = 128
_PT_W = _CHUNK_PTS * 6
_NCH = (_NPTS // 2) // _CHUNK_PTS
_GRP = _CHUNK_PTS // 16

_SUP_VOX = 512
_NSUP = 62464 // _SUP_VOX
_NSUP_MAIN = 120
_EXP_VOX = 256
_EXP_W = _EXP_VOX * 4
_EGRP = _EXP_VOX // 16


def _body(pts_hbm, val_hbm, out_hbm,
          ptsb0, ptsb1, valb0, valb1, occ_v, exp0, exp1,
          imp0, imp1, imp2, imp3, xch,
          isem0, isem1, msem0, msem1, msem2, msem3, osem0, osem1):
    c = lax.axis_index("c")
    s = lax.axis_index("s")
    k = s // 2
    h = s % 2
    b = c * 8 + k

    ptsb = (ptsb0, ptsb1)
    valb = (valb0, valb1)
    expb = (exp0, exp1)
    impb = (imp0, imp1, imp2, imp3)
    isem = (isem0, isem1)
    msem = (msem0, msem1, msem2, msem3)
    osem = (osem0, osem1)

    iota = lax.iota(jnp.int32, 16)
    iota4 = iota * 4
    iota6 = iota * 6
    ones = jnp.full((16,), 1.0, jnp.float32)
    zeros = jnp.zeros((16,), jnp.float32)

    gid = c * 16 + s
    pgid = gid + 1 - 2 * h
    exp_slot = gid * _HALF_VOX
    imp_slot = pgid * _HALF_VOX

    pts_base = b * (_NPTS * 6) + h * (_NCH * _PT_W)
    val_base = b * _NPTS + h * (_NCH * _CHUNK_PTS)

    def start_in(ci, buf):
        pltpu.async_copy(
            pts_hbm.at[pl.ds(pts_base + ci * _PT_W, _PT_W)],
            ptsb[buf], isem[buf])
        pltpu.async_copy(
            val_hbm.at[pl.ds(val_base + ci * _CHUNK_PTS, _CHUNK_PTS)],
            valb[buf], isem[buf])

    def wait_in(buf):
        pltpu.make_async_copy(
            pts_hbm.at[pl.ds(0, _PT_W)], ptsb[buf], isem[buf]).wait()
        pltpu.make_async_copy(
            val_hbm.at[pl.ds(0, _CHUNK_PTS)], valb[buf], isem[buf]).wait()

    start_in(0, 0)
    start_in(1, 1)

    @pl.loop(0, _NVOX_PAD // 16, unroll=8)
    def _zero(i):
        occ_v[pl.ds(i * 16, 16)] = zeros

    def process_chunk(buf):
        pv = ptsb[buf]
        vv = valb[buf]
        for g in range(_GRP):
            gx = iota6 + (g * 96)
            x = plsc.load_gather(pv, [gx])
            y = plsc.load_gather(pv, [gx + 1])
            z = plsc.load_gather(pv, [gx + 2])
            vf = vv[pl.ds(g * 16, 16)]
            msk = vf > 0.0

            def vox(p):
                t = (p + 2.0) * 12.25
                t = jnp.minimum(jnp.maximum(t, 0.0), 49.0)
                return t.astype(jnp.int32)

            v = (vox(x) * 50 + vox(y)) * 50 + vox(z)
            plsc.store_scatter(occ_v, [v], ones, mask=msk)

    @pl.loop(0, _NCH, step=2)
    def _chunks(e):
        for bu in (0, 1):
            ci = e + bu
            wait_in(bu)
            process_chunk(bu)
            nxt = ci + 2

            @pl.when(nxt < _NCH)
            def _():
                start_in(nxt, bu)

    pltpu.sync_copy(occ_v.at[pl.ds((1 - h) * _HALF_VOX, _HALF_VOX)],
                    xch.at[pl.ds(exp_slot, _HALF_VOX)])
    plsc.subcore_barrier()

    vstart = h * _HALF_VOX
    out_base = b * _OUT_W + vstart * 4

    def start_imp(e, buf):
        pltpu.async_copy(xch.at[pl.ds(imp_slot + e * _SUP_VOX, _SUP_VOX)],
                         impb[buf], msem[buf])

    def wait_imp(buf):
        pltpu.make_async_copy(xch.at[pl.ds(0, _SUP_VOX)],
                              impb[buf], msem[buf]).wait()

    def start_out(eu, buf):
        pltpu.async_copy(expb[buf],
                         out_hbm.at[pl.ds(out_base + eu * _EXP_W, _EXP_W)],
                         osem[buf])

    def wait_out(buf):
        pltpu.make_async_copy(expb[buf],
                              out_hbm.at[pl.ds(0, _EXP_W)], osem[buf]).wait()

    @pl.loop(0, _EXP_W // 16, unroll=8)
    def _zexp(i):
        exp0[pl.ds(i * 16, 16)] = zeros
        exp1[pl.ds(i * 16, 16)] = zeros

    for i in range(4):
        start_imp(i, i)

    def do_sup(ei, bu, guard_out):
        wait_imp(bu)
        im = impb[bu]
        for sub in (0, 1):
            eu = ei * 2 + sub

            if guard_out:
                @pl.when(eu >= 2)
                def _():
                    wait_out(sub)
            else:
                wait_out(sub)

            ex = expb[sub]
            for g in range(_EGRP):
                off = sub * _EXP_VOX + g * 16
                own = occ_v[pl.ds(vstart + ei * _SUP_VOX + off, 16)]
                other = im[pl.ds(off, 16)]
                plsc.store_scatter(ex, [iota4 + g * 64],
                                   jnp.maximum(own, other))
            start_out(eu, sub)

    @pl.loop(0, _NSUP_MAIN, step=4)
    def _sup(e):
        for bu in range(4):
            ei = e + bu
            do_sup(ei, bu, guard_out=True)
            nxt = ei + 4

            @pl.when(nxt < _NSUP)
            def _():
                start_imp(nxt, bu)

    do_sup(_NSUP_MAIN, 0, guard_out=False)
    do_sup(_NSUP_MAIN + 1, 1, guard_out=False)
    wait_out(0)
    wait_out(1)

    tail_vbase = _NSUP * _SUP_VOX
    tgrps = jnp.where(h == 0, 3, 2)
    pltpu.sync_copy(xch.at[pl.ds(imp_slot + tail_vbase, 48)],
                    imp0.at[pl.ds(0, 48)])

    @pl.loop(0, tgrps)
    def _tgrp(g):
        own = occ_v[pl.ds(vstart + tail_vbase + g * 16, 16)]
        other = imp0[pl.ds(g * 16, 16)]
        plsc.store_scatter(exp0, [iota4 + g * 64], jnp.maximum(own, other))

    tout = out_base + tail_vbase * 4

    @pl.when(h == 0)
    def _():
        pltpu.sync_copy(exp0.at[pl.ds(0, 192)],
                        out_hbm.at[pl.ds(tout, 192)])

    @pl.when(h == 1)
    def _():
        pltpu.sync_copy(exp0.at[pl.ds(0, 96)],
                        out_hbm.at[pl.ds(tout, 96)])


@jax.jit
def _voxelize(pts_flat, val_flat):
    mesh = plsc.VectorSubcoreMesh(core_axis_name="c", subcore_axis_name="s")
    return pl.kernel(
        _body,
        out_type=jax.ShapeDtypeStruct((_B * _OUT_W,), jnp.float32),
        mesh=mesh,
        compiler_params=pltpu.CompilerParams(needs_layout_passes=False),
        scratch_types=[
            pltpu.VMEM((_PT_W,), jnp.float32),
            pltpu.VMEM((_PT_W,), jnp.float32),
            pltpu.VMEM((_CHUNK_PTS,), jnp.float32),
            pltpu.VMEM((_CHUNK_PTS,), jnp.float32),
            pltpu.VMEM((_NVOX_PAD,), jnp.float32),
            pltpu.VMEM((_EXP_W,), jnp.float32),
            pltpu.VMEM((_EXP_W,), jnp.float32),
            pltpu.VMEM((_SUP_VOX,), jnp.float32),
            pltpu.VMEM((_SUP_VOX,), jnp.float32),
            pltpu.VMEM((_SUP_VOX,), jnp.float32),
            pltpu.VMEM((_SUP_VOX,), jnp.float32),
            pltpu.HBM((32 * _HALF_VOX,), jnp.float32),
            pltpu.SemaphoreType.DMA,
            pltpu.SemaphoreType.DMA,
            pltpu.SemaphoreType.DMA,
            pltpu.SemaphoreType.DMA,
            pltpu.SemaphoreType.DMA,
            pltpu.SemaphoreType.DMA,
            pltpu.SemaphoreType.DMA,
            pltpu.SemaphoreType.DMA,
        ],
    )(pts_flat, val_flat)


def kernel(pointclouds, valid_points):
    pts_flat = pointclouds.reshape(-1)
    val_flat = valid_points.astype(jnp.float32).reshape(-1)
    out = _voxelize(pts_flat, val_flat)
    return out.reshape(_B, _RES, _RES, _RES, 4)
